# baseline (speedup 1.0000x reference)
import functools

import jax
import jax.numpy as jnp
from jax.experimental import pallas as pl

N_IN = 2
EMB = 128
B = 16384
D = 128
ROWS = 1024


def _fused_body(x1, x2, w1, b1, w2, b2, w3, b3, sel, wb,
                out_ref, out1_ref, out2_ref, ws_ref):
    a10 = x1[0]
    a11 = x1[1]
    a20 = x2[0]
    a21 = x2[1]

    def dock(x, w, b, i):
        return jax.nn.relu(
            jnp.dot(x.astype(jnp.bfloat16), w[i].astype(jnp.bfloat16),
                    preferred_element_type=jnp.float32) + b[i:i + 1, :])

    s1 = sel[0:1, :]
    o1 = jnp.where(s1 == 0, dock(a10, w1, b1, 0), dock(a11, w1, b1, 1))
    s2 = sel[1:2, :]
    o2 = jnp.where(s2 == 0, dock(a10, w2, b2, 0), dock(a11, w2, b2, 1))

    ws = a20 * wb[0:1, :] + a21 * wb[1:2, :]

    s3 = sel[2:3, :]
    m = jnp.where(s3 == 0, dock(a20, w3, b3, 0),
        jnp.where(s3 == 1, dock(a21, w3, b3, 1),
        jnp.where(s3 == 2, dock(o1, w3, b3, 2),
        jnp.where(s3 == 3, dock(o2, w3, b3, 3), dock(ws, w3, b3, 4)))))

    out_ref[...] = m
    out1_ref[...] = o1
    out2_ref[...] = o2
    ws_ref[...] = ws


def kernel(outputs1, outputs2, available, W1, b1, W2, b2, W3, b3, ws_w):
    del available

    k = jax.random.key(42)
    k1, k2, k3 = jax.random.split(k, 3)
    ones12 = jnp.ones((1, N_IN), dtype=jnp.float32)
    p12 = ones12 / jnp.sum(ones12, axis=-1, keepdims=True)
    idx1 = jax.random.categorical(k1, jnp.log(p12), shape=(1, EMB))
    idx2 = jax.random.categorical(k2, jnp.log(p12), shape=(1, EMB))
    avail = jnp.ones((1, N_IN + 3), dtype=jnp.float32)
    p3 = avail / jnp.sum(avail, axis=-1, keepdims=True)
    idx3 = jax.random.categorical(k3, jnp.log(p3), shape=(1, EMB))
    sel = jnp.concatenate([idx1, idx2, idx3], axis=0).astype(jnp.int32)

    w = ws_w * avail[0, :N_IN]
    w = w / jnp.sum(w)
    wb = jnp.broadcast_to(w[:, None], (N_IN, EMB)).astype(jnp.float32)

    grid = (B // ROWS,)
    row_spec = pl.BlockSpec((ROWS, D), lambda i: (i, 0))
    xin_spec = pl.BlockSpec((N_IN, ROWS, D), lambda i: (0, i, 0))
    full = lambda shape: pl.BlockSpec(shape, lambda i: (0,) * len(shape))

    out_shapes = tuple(
        jax.ShapeDtypeStruct((B, EMB), jnp.float32) for _ in range(4))

    out, out1, out2, wsout = pl.pallas_call(
        _fused_body,
        grid=grid,
        in_specs=[
            xin_spec, xin_spec,
            full((N_IN, D, EMB)), full((N_IN, EMB)),
            full((N_IN, D, EMB)), full((N_IN, EMB)),
            full((N_IN + 3, D, EMB)), full((N_IN + 3, EMB)),
            full((3, EMB)), full((N_IN, EMB)),
        ],
        out_specs=(row_spec, row_spec, row_spec, row_spec),
        out_shape=out_shapes,
    )(outputs1, outputs2, W1, b1, W2, b2, W3, b3, sel, wb)

    return (out, (out1, out2, wsout))


# speedup vs baseline: 1.1148x; 1.1148x over previous
import functools

import jax
import jax.numpy as jnp
from jax.experimental import pallas as pl

N_IN = 2
EMB = 128
B = 16384
D = 128
ROWS = 4096


def _fused_body(x1, x2, w1, b1, w2, b2, w3, b3, sel, wb,
                out_ref, out1_ref, out2_ref, ws_ref):
    a10 = x1[0]
    a11 = x1[1]
    a20 = x2[0]
    a21 = x2[1]

    def dock(x, w, b, i):
        return jax.nn.relu(
            jnp.dot(x.astype(jnp.bfloat16), w[i].astype(jnp.bfloat16),
                    preferred_element_type=jnp.float32) + b[i:i + 1, :])

    s1 = sel[0:1, :]
    o1 = jnp.where(s1 == 0, dock(a10, w1, b1, 0), dock(a11, w1, b1, 1))
    s2 = sel[1:2, :]
    o2 = jnp.where(s2 == 0, dock(a10, w2, b2, 0), dock(a11, w2, b2, 1))

    ws = a20 * wb[0:1, :] + a21 * wb[1:2, :]

    s3 = sel[2:3, :]
    m = jnp.where(s3 == 0, dock(a20, w3, b3, 0),
        jnp.where(s3 == 1, dock(a21, w3, b3, 1),
        jnp.where(s3 == 2, dock(o1, w3, b3, 2),
        jnp.where(s3 == 3, dock(o2, w3, b3, 3), dock(ws, w3, b3, 4)))))

    out_ref[...] = m
    out1_ref[...] = o1
    out2_ref[...] = o2
    ws_ref[...] = ws


def kernel(outputs1, outputs2, available, W1, b1, W2, b2, W3, b3, ws_w):
    del available

    k = jax.random.key(42)
    k1, k2, k3 = jax.random.split(k, 3)
    ones12 = jnp.ones((1, N_IN), dtype=jnp.float32)
    p12 = ones12 / jnp.sum(ones12, axis=-1, keepdims=True)
    idx1 = jax.random.categorical(k1, jnp.log(p12), shape=(1, EMB))
    idx2 = jax.random.categorical(k2, jnp.log(p12), shape=(1, EMB))
    avail = jnp.ones((1, N_IN + 3), dtype=jnp.float32)
    p3 = avail / jnp.sum(avail, axis=-1, keepdims=True)
    idx3 = jax.random.categorical(k3, jnp.log(p3), shape=(1, EMB))
    sel = jnp.concatenate([idx1, idx2, idx3], axis=0).astype(jnp.int32)

    w = ws_w * avail[0, :N_IN]
    w = w / jnp.sum(w)
    wb = jnp.broadcast_to(w[:, None], (N_IN, EMB)).astype(jnp.float32)

    grid = (B // ROWS,)
    row_spec = pl.BlockSpec((ROWS, D), lambda i: (i, 0))
    xin_spec = pl.BlockSpec((N_IN, ROWS, D), lambda i: (0, i, 0))
    full = lambda shape: pl.BlockSpec(shape, lambda i: (0,) * len(shape))

    out_shapes = tuple(
        jax.ShapeDtypeStruct((B, EMB), jnp.float32) for _ in range(4))

    out, out1, out2, wsout = pl.pallas_call(
        _fused_body,
        grid=grid,
        in_specs=[
            xin_spec, xin_spec,
            full((N_IN, D, EMB)), full((N_IN, EMB)),
            full((N_IN, D, EMB)), full((N_IN, EMB)),
            full((N_IN + 3, D, EMB)), full((N_IN + 3, EMB)),
            full((3, EMB)), full((N_IN, EMB)),
        ],
        out_specs=(row_spec, row_spec, row_spec, row_spec),
        out_shape=out_shapes,
    )(outputs1, outputs2, W1, b1, W2, b2, W3, b3, sel, wb)

    return (out, (out1, out2, wsout))
